# baseline (speedup 1.0000x reference)
import functools

import jax
import jax.numpy as jnp
from jax import lax
from jax.experimental import pallas as pl
from jax.experimental.pallas import tpu as pltpu
from jax.experimental.pallas import tpu_sc as plsc

B = 32
EMB = 32
N_UID = 1000000
TOPK = 50000.0
RQ = 10000
NB = (N_UID // 4) // RQ
K1 = 12
K2 = 8

_P1 = (0.15, 0.3, 0.45, 0.6, 0.75, 0.9,
       1.05, 1.2, 1.35, 1.5, 1.9, 2.3)


def _key4(xr, w, tbl):
    qt = lax.dot_general(w, xr, (((1,), (1,)), ((), ())),
                         preferred_element_type=jnp.float32)
    qt4 = jnp.tile(qt, (4, 4))
    r = lax.broadcasted_iota(jnp.int32, (128, 128), 0)
    c = lax.broadcasted_iota(jnp.int32, (128, 128), 1)
    qtb = jnp.where((r // 32) == (c // 32), qt4, 0.0).astype(jnp.bfloat16)
    s4 = lax.dot_general(tbl, qtb, (((1,), (0,)), ((), ())),
                         preferred_element_type=jnp.float32)
    return jnp.abs(s4 - 1.0)


def _bracket_step(T, C):
    below = (C < TOPK).astype(jnp.float32)
    nbv = jnp.sum(below, axis=0, keepdims=True)
    kidx = lax.broadcasted_iota(jnp.int32, T.shape, 0).astype(jnp.float32)
    wlo = (kidx == (nbv - 1.0)).astype(jnp.float32)
    whi = (kidx == nbv).astype(jnp.float32)
    lo = jnp.sum(T * wlo, axis=0, keepdims=True)
    clo = jnp.sum(C * wlo, axis=0, keepdims=True)
    hi = jnp.sum(T * whi, axis=0, keepdims=True)
    chi = jnp.sum(C * whi, axis=0, keepdims=True)
    return lo, clo, hi, chi


def _grpsum(xkk):
    parts = [xkk[:, 32 * g:32 * g + 32] for g in range(4)]
    s = (parts[0] + parts[1]) + (parts[2] + parts[3])
    return jnp.concatenate([s, s, s, s], axis=1)


def _grpmax(row):
    parts = [row[:, 32 * g:32 * g + 32] for g in range(4)]
    m = jnp.maximum(jnp.maximum(parts[0], parts[1]),
                    jnp.maximum(parts[2], parts[3]))
    return jnp.concatenate([m, m, m, m], axis=1)


def _count_rows(key, thr_ref, K):
    rows = [jnp.sum((key <= thr_ref[k:k + 1, :]).astype(jnp.float32),
                    axis=0, keepdims=True) for k in range(K)]
    return jnp.concatenate(rows, axis=0)


def _mega_body(xr_ref, w_ref, thr1_ref, aux_ref, rows4_ref, tbl_ref, out_ref,
               thr_s, st_s, cnt_s, acc_s, gap_s):
    p = pl.program_id(0)
    i = pl.program_id(1)
    nb = pl.num_programs(1)
    key = _key4(xr_ref[...], w_ref[...], tbl_ref[...])
    ramp8 = (lax.broadcasted_iota(jnp.int32, (K2, 128), 0)
             .astype(jnp.float32) + 1.0) * (1.0 / (K2 + 1.0))

    @pl.when(i == 0)
    def _():
        cnt_s[...] = jnp.zeros_like(cnt_s)

    @pl.when(jnp.logical_and(p == 0, i == 0))
    def _():
        st_s[...] = jnp.zeros_like(st_s)

    @pl.when(p == 0)
    def _():
        cnt_s[...] = cnt_s[...] + _count_rows(key, thr1_ref, K1)
        st_s[5:6, :] = jnp.maximum(st_s[5:6, :],
                                   jnp.max(key, axis=0, keepdims=True))

        @pl.when(i == nb - 1)
        def _():
            z = jnp.zeros((1, 128), jnp.float32)
            T = jnp.concatenate([z, thr1_ref[...], _grpmax(st_s[5:6, :])],
                                axis=0)
            C = jnp.concatenate([z, _grpsum(cnt_s[...]),
                                 jnp.full((1, 128), float(N_UID),
                                          jnp.float32)], axis=0)
            lo, clo, hi, chi = _bracket_step(T, C)
            st_s[0:1, :] = lo
            st_s[1:2, :] = hi
            st_s[2:3, :] = clo
            st_s[3:4, :] = chi
            thr_s[...] = lo + (hi - lo) * ramp8

    @pl.when(jnp.logical_or(p == 1, p == 2))
    def _():
        cnt_s[0:K2, :] = cnt_s[0:K2, :] + _count_rows(key, thr_s, K2)

        @pl.when(i == nb - 1)
        def _():
            T = jnp.concatenate([st_s[0:1, :], thr_s[...], st_s[1:2, :]],
                                axis=0)
            C = jnp.concatenate([st_s[2:3, :], _grpsum(cnt_s[0:K2, :]),
                                 st_s[3:4, :]], axis=0)
            lo, clo, hi, chi = _bracket_step(T, C)
            st_s[0:1, :] = lo
            st_s[1:2, :] = hi
            st_s[2:3, :] = clo
            st_s[3:4, :] = chi
            thr_s[...] = lo + (hi - lo) * ramp8

    @pl.when(p == 3)
    def _():
        @pl.when(i == 0)
        def _():
            acc_s[...] = jnp.zeros_like(acc_s)
            gap_s[...] = jnp.zeros_like(gap_s)

        mlo = (key <= st_s[0:1, :]).astype(jnp.bfloat16)
        mhi = (key <= st_s[1:2, :]).astype(jnp.bfloat16)
        mgap = mhi - mlo
        tbl = tbl_ref[...]
        acc_s[...] = acc_s[...] + lax.dot_general(
            mlo, tbl, (((0,), (0,)), ((), ())),
            preferred_element_type=jnp.float32)
        gap_s[...] = gap_s[...] + lax.dot_general(
            mgap, tbl, (((0,), (0,)), ((), ())),
            preferred_element_type=jnp.float32)

        @pl.when(i == nb - 1)
        def _():
            fr_row = (TOPK - st_s[2:3, :]) \
                / jnp.maximum(st_s[3:4, :] - st_s[2:3, :], 1.0)
            r = lax.broadcasted_iota(jnp.int32, (128, 128), 0)
            c = lax.broadcasted_iota(jnp.int32, (128, 128), 1)
            eye = (r == c).astype(jnp.float32)
            fr_col = lax.dot_general(eye, fr_row, (((1,), (1,)), ((), ())),
                                     preferred_element_type=jnp.float32)
            tot128 = acc_s[...] + fr_col * gap_s[...]
            tot = jnp.zeros((B, EMB), jnp.float32)
            for g in range(4):
                sl = slice(32 * g, 32 * g + 32)
                tot = tot + tot128[sl, sl]
            outc = jnp.zeros((B, 1), jnp.float32)
            for m in range(4):
                pr = jnp.sum(tot * rows4_ref[:, 32 * m:32 * m + 32],
                             axis=1, keepdims=True)
                outc = outc + aux_ref[:, 1 + m:2 + m] * pr
            out_ref[...] = jnp.broadcast_to(outc * (1.0 / TOPK),
                                            out_ref.shape)


def _mega_pass(xr, w, thr1, aux, rows4, tbl4):
    return pl.pallas_call(
        _mega_body,
        grid=(4, NB),
        in_specs=[
            pl.BlockSpec((B, EMB), lambda p, i: (0, 0)),
            pl.BlockSpec((EMB, EMB), lambda p, i: (0, 0)),
            pl.BlockSpec((K1, 128), lambda p, i: (0, 0)),
            pl.BlockSpec((B, 8), lambda p, i: (0, 0)),
            pl.BlockSpec((B, 128), lambda p, i: (0, 0)),
            pl.BlockSpec((RQ, 128), lambda p, i: (i, 0)),
        ],
        out_specs=pl.BlockSpec((B, 8), lambda p, i: (0, 0)),
        out_shape=jax.ShapeDtypeStruct((B, 8), jnp.float32),
        scratch_shapes=[
            pltpu.VMEM((K2, 128), jnp.float32),
            pltpu.VMEM((8, 128), jnp.float32),
            pltpu.VMEM((K1, 128), jnp.float32),
            pltpu.VMEM((128, 128), jnp.float32),
            pltpu.VMEM((128, 128), jnp.float32),
        ],
    )(xr, w, thr1, aux, rows4, tbl4)


def _sc_gather(pidx, table4):
    mesh = plsc.VectorSubcoreMesh(core_axis_name="c", subcore_axis_name="s")

    @functools.partial(
        pl.kernel,
        mesh=mesh,
        out_type=jax.ShapeDtypeStruct((B, 128), jnp.float32),
        scratch_types=[
            pltpu.VMEM((B,), jnp.int32),
            pltpu.VMEM((B, 128), jnp.float32),
            pltpu.SemaphoreType.DMA,
        ],
    )
    def k(idx_hbm, tbl_hbm, out_hbm, idx_v, rows_v, sem):
        wid = lax.axis_index("s") * 2 + lax.axis_index("c")

        @pl.when(wid == 0)
        def _():
            pltpu.sync_copy(idx_hbm, idx_v)
            pltpu.async_copy(tbl_hbm.at[idx_v], rows_v, sem).wait()
            pltpu.sync_copy(rows_v, out_hbm)

    return k(pidx, table4)


def kernel(x, tgt_uid_table, tgt_iid_table, W_rp):
    iid = x[:, 0].astype(jnp.int32)
    xr = x[:, 1:EMB + 1]
    tbl4 = tgt_uid_table.reshape(N_UID // 4, 128).astype(jnp.bfloat16)
    itbl4 = tgt_iid_table.reshape(-1, 128)

    rows4 = _sc_gather(iid // 4, itbl4)
    sel = (iid % 4)[:, None] == jnp.arange(4)[None, :]

    t1 = jnp.asarray(_P1, jnp.float32)
    thr1 = jnp.tile(jnp.broadcast_to(t1[:, None], (K1, B)), (1, 4))
    aux = jnp.zeros((B, 8), jnp.float32)
    aux = aux.at[:, 1:5].set(sel.astype(jnp.float32))

    out32 = _mega_pass(xr, W_rp, thr1, aux, rows4, tbl4)
    return out32[:, 0]


# speedup vs baseline: 1.0148x; 1.0148x over previous
import functools

import jax
import jax.numpy as jnp
from jax import lax
from jax.experimental import pallas as pl
from jax.experimental.pallas import tpu as pltpu
from jax.experimental.pallas import tpu_sc as plsc

B = 32
EMB = 32
N_UID = 1000000
TOPK = 50000.0
RQ = 10000
NB = (N_UID // 4) // RQ
K1 = 12
K2 = 8

_P1 = (0.15, 0.3, 0.45, 0.6, 0.75, 0.9,
       1.05, 1.2, 1.35, 1.5, 1.9, 2.3)


def _key4(xr, w, tbl):
    qt = lax.dot_general(w, xr, (((1,), (1,)), ((), ())),
                         preferred_element_type=jnp.float32)
    qt4 = jnp.tile(qt, (4, 4))
    r = lax.broadcasted_iota(jnp.int32, (128, 128), 0)
    c = lax.broadcasted_iota(jnp.int32, (128, 128), 1)
    qtb = jnp.where((r // 32) == (c // 32), qt4, 0.0)
    s4 = lax.dot_general(tbl, qtb, (((1,), (0,)), ((), ())),
                         preferred_element_type=jnp.float32)
    return jnp.abs(s4 - 1.0)


def _bracket_step(T, C):
    below = (C < TOPK).astype(jnp.float32)
    nbv = jnp.sum(below, axis=0, keepdims=True)
    kidx = lax.broadcasted_iota(jnp.int32, T.shape, 0).astype(jnp.float32)
    wlo = (kidx == (nbv - 1.0)).astype(jnp.float32)
    whi = (kidx == nbv).astype(jnp.float32)
    lo = jnp.sum(T * wlo, axis=0, keepdims=True)
    clo = jnp.sum(C * wlo, axis=0, keepdims=True)
    hi = jnp.sum(T * whi, axis=0, keepdims=True)
    chi = jnp.sum(C * whi, axis=0, keepdims=True)
    return lo, clo, hi, chi


def _grpsum(xkk):
    parts = [xkk[:, 32 * g:32 * g + 32] for g in range(4)]
    s = (parts[0] + parts[1]) + (parts[2] + parts[3])
    return jnp.concatenate([s, s, s, s], axis=1)


def _grpmax(row):
    parts = [row[:, 32 * g:32 * g + 32] for g in range(4)]
    m = jnp.maximum(jnp.maximum(parts[0], parts[1]),
                    jnp.maximum(parts[2], parts[3]))
    return jnp.concatenate([m, m, m, m], axis=1)


def _count_rows(key, thr_ref, K):
    rows = [jnp.sum((key <= thr_ref[k:k + 1, :]).astype(jnp.float32),
                    axis=0, keepdims=True) for k in range(K)]
    return jnp.concatenate(rows, axis=0)


def _mega_body(xr_ref, w_ref, thr1_ref, aux_ref, rows4_ref, tbl_ref, out_ref,
               thr_s, st_s, cnt_s, acc_s, gap_s):
    p = pl.program_id(0)
    i = pl.program_id(1)
    nb = pl.num_programs(1)
    key = _key4(xr_ref[...], w_ref[...], tbl_ref[...])
    ramp8 = (lax.broadcasted_iota(jnp.int32, (K2, 128), 0)
             .astype(jnp.float32) + 1.0) * (1.0 / (K2 + 1.0))

    @pl.when(i == 0)
    def _():
        cnt_s[...] = jnp.zeros_like(cnt_s)

    @pl.when(jnp.logical_and(p == 0, i == 0))
    def _():
        st_s[...] = jnp.zeros_like(st_s)

    @pl.when(p == 0)
    def _():
        cnt_s[...] = cnt_s[...] + _count_rows(key, thr1_ref, K1)
        st_s[5:6, :] = jnp.maximum(st_s[5:6, :],
                                   jnp.max(key, axis=0, keepdims=True))

        @pl.when(i == nb - 1)
        def _():
            z = jnp.zeros((1, 128), jnp.float32)
            T = jnp.concatenate([z, thr1_ref[...], _grpmax(st_s[5:6, :])],
                                axis=0)
            C = jnp.concatenate([z, _grpsum(cnt_s[...]),
                                 jnp.full((1, 128), float(N_UID),
                                          jnp.float32)], axis=0)
            lo, clo, hi, chi = _bracket_step(T, C)
            st_s[0:1, :] = lo
            st_s[1:2, :] = hi
            st_s[2:3, :] = clo
            st_s[3:4, :] = chi
            thr_s[...] = lo + (hi - lo) * ramp8

    @pl.when(jnp.logical_or(p == 1, p == 2))
    def _():
        cnt_s[0:K2, :] = cnt_s[0:K2, :] + _count_rows(key, thr_s, K2)

        @pl.when(i == nb - 1)
        def _():
            T = jnp.concatenate([st_s[0:1, :], thr_s[...], st_s[1:2, :]],
                                axis=0)
            C = jnp.concatenate([st_s[2:3, :], _grpsum(cnt_s[0:K2, :]),
                                 st_s[3:4, :]], axis=0)
            lo, clo, hi, chi = _bracket_step(T, C)
            st_s[0:1, :] = lo
            st_s[1:2, :] = hi
            st_s[2:3, :] = clo
            st_s[3:4, :] = chi
            thr_s[...] = lo + (hi - lo) * ramp8

    @pl.when(p == 3)
    def _():
        @pl.when(i == 0)
        def _():
            acc_s[...] = jnp.zeros_like(acc_s)
            gap_s[...] = jnp.zeros_like(gap_s)

        mlo = (key <= st_s[0:1, :]).astype(jnp.float32)
        mhi = (key <= st_s[1:2, :]).astype(jnp.float32)
        mgap = mhi - mlo
        tbl = tbl_ref[...]
        acc_s[...] = acc_s[...] + lax.dot_general(
            mlo, tbl, (((0,), (0,)), ((), ())),
            preferred_element_type=jnp.float32)
        gap_s[...] = gap_s[...] + lax.dot_general(
            mgap, tbl, (((0,), (0,)), ((), ())),
            preferred_element_type=jnp.float32)

        @pl.when(i == nb - 1)
        def _():
            fr_row = (TOPK - st_s[2:3, :]) \
                / jnp.maximum(st_s[3:4, :] - st_s[2:3, :], 1.0)
            r = lax.broadcasted_iota(jnp.int32, (128, 128), 0)
            c = lax.broadcasted_iota(jnp.int32, (128, 128), 1)
            eye = (r == c).astype(jnp.float32)
            fr_col = lax.dot_general(eye, fr_row, (((1,), (1,)), ((), ())),
                                     preferred_element_type=jnp.float32)
            tot128 = acc_s[...] + fr_col * gap_s[...]
            tot = jnp.zeros((B, EMB), jnp.float32)
            for g in range(4):
                sl = slice(32 * g, 32 * g + 32)
                tot = tot + tot128[sl, sl]
            outc = jnp.zeros((B, 1), jnp.float32)
            for m in range(4):
                pr = jnp.sum(tot * rows4_ref[:, 32 * m:32 * m + 32],
                             axis=1, keepdims=True)
                outc = outc + aux_ref[:, 1 + m:2 + m] * pr
            out_ref[...] = jnp.broadcast_to(outc * (1.0 / TOPK),
                                            out_ref.shape)


def _mega_pass(xr, w, thr1, aux, rows4, tbl4):
    return pl.pallas_call(
        _mega_body,
        grid=(4, NB),
        in_specs=[
            pl.BlockSpec((B, EMB), lambda p, i: (0, 0)),
            pl.BlockSpec((EMB, EMB), lambda p, i: (0, 0)),
            pl.BlockSpec((K1, 128), lambda p, i: (0, 0)),
            pl.BlockSpec((B, 8), lambda p, i: (0, 0)),
            pl.BlockSpec((B, 128), lambda p, i: (0, 0)),
            pl.BlockSpec((RQ, 128), lambda p, i: (i, 0)),
        ],
        out_specs=pl.BlockSpec((B, 8), lambda p, i: (0, 0)),
        out_shape=jax.ShapeDtypeStruct((B, 8), jnp.float32),
        scratch_shapes=[
            pltpu.VMEM((K2, 128), jnp.float32),
            pltpu.VMEM((8, 128), jnp.float32),
            pltpu.VMEM((K1, 128), jnp.float32),
            pltpu.VMEM((128, 128), jnp.float32),
            pltpu.VMEM((128, 128), jnp.float32),
        ],
    )(xr, w, thr1, aux, rows4, tbl4)


def _sc_gather(pidx, table4):
    mesh = plsc.VectorSubcoreMesh(core_axis_name="c", subcore_axis_name="s")

    @functools.partial(
        pl.kernel,
        mesh=mesh,
        out_type=jax.ShapeDtypeStruct((B, 128), jnp.float32),
        scratch_types=[
            pltpu.VMEM((B,), jnp.int32),
            pltpu.VMEM((B, 128), jnp.float32),
            pltpu.SemaphoreType.DMA,
        ],
    )
    def k(idx_hbm, tbl_hbm, out_hbm, idx_v, rows_v, sem):
        wid = lax.axis_index("s") * 2 + lax.axis_index("c")

        @pl.when(wid == 0)
        def _():
            pltpu.sync_copy(idx_hbm, idx_v)
            pltpu.async_copy(tbl_hbm.at[idx_v], rows_v, sem).wait()
            pltpu.sync_copy(rows_v, out_hbm)

    return k(pidx, table4)


def kernel(x, tgt_uid_table, tgt_iid_table, W_rp):
    iid = x[:, 0].astype(jnp.int32)
    xr = x[:, 1:EMB + 1]
    tbl4 = tgt_uid_table.reshape(N_UID // 4, 128)
    itbl4 = tgt_iid_table.reshape(-1, 128)

    rows4 = _sc_gather(iid // 4, itbl4)
    sel = (iid % 4)[:, None] == jnp.arange(4)[None, :]

    t1 = jnp.asarray(_P1, jnp.float32)
    thr1 = jnp.tile(jnp.broadcast_to(t1[:, None], (K1, B)), (1, 4))
    aux = jnp.zeros((B, 8), jnp.float32)
    aux = aux.at[:, 1:5].set(sel.astype(jnp.float32))

    out32 = _mega_pass(xr, W_rp, thr1, aux, rows4, tbl4)
    return out32[:, 0]
